# baseline (speedup 1.0000x reference)
import functools

import jax
import jax.numpy as jnp
from jax import lax
from jax.experimental import pallas as pl
from jax.experimental.pallas import tpu as pltpu
from jax.experimental.pallas import tpu_sc as plsc

N_NODES = 10000
N_EDGES = 320000
FEATS = 128
HIDDEN = 64
CLASSES = 64
ALPHA = 0.1
DEPTH = 10

NC = 2
NS = 16
NW = NC * NS
CHUNK = 128
NCHUNKS = 80
EDGES_PER_TILE = NCHUNKS * CHUNK
EDGES_PAD = NW * EDGES_PER_TILE
NP = 10240
W = CLASSES
ROWS_PER_TILE = NP // NS
HT_ROWS = NP * W // 128
TROWS_PER_TILE = HT_ROWS // NS
RR = 128
NROUNDS = ROWS_PER_TILE // RR

MSLOTS = 4
ISLOTS = 8

_MESH = plsc.VectorSubcoreMesh(core_axis_name="c", subcore_axis_name="s")
_UNTILED = pltpu.CompilerParams(use_tc_tiling_on_sc=False)


def _front_body(x_ref, w1_ref, b1_ref, w2_ref, b2_ref, o_ref):
    h = jnp.dot(x_ref[...], w1_ref[...], preferred_element_type=jnp.float32)
    h = jnp.maximum(h + b1_ref[...], 0.0)
    h = jnp.dot(h, w2_ref[...], preferred_element_type=jnp.float32) + b2_ref[...]
    o_ref[...] = jnp.concatenate(
        [h, jnp.zeros((h.shape[0], 128 - CLASSES), jnp.float32)], axis=1
    )


def _front(xp, W1, b1, W2, b2):
    R = 1024
    return pl.pallas_call(
        _front_body,
        grid=(NP // R,),
        in_specs=[
            pl.BlockSpec((R, FEATS), lambda i: (i, 0)),
            pl.BlockSpec((FEATS, HIDDEN), lambda i: (0, 0)),
            pl.BlockSpec((1, HIDDEN), lambda i: (0, 0)),
            pl.BlockSpec((HIDDEN, CLASSES), lambda i: (0, 0)),
            pl.BlockSpec((1, CLASSES), lambda i: (0, 0)),
        ],
        out_specs=pl.BlockSpec((R, 128), lambda i: (i, 0)),
        out_shape=jax.ShapeDtypeStruct((NP, 128), jnp.float32),
    )(xp, W1, b1.reshape(1, HIDDEN), W2, b2.reshape(1, CLASSES))


@functools.partial(
    pl.kernel,
    out_type=(
        jax.ShapeDtypeStruct((NP, W), jnp.float32),
        jax.ShapeDtypeStruct((NP, W), jnp.float32),
    ),
    mesh=_MESH,
    scratch_types=[
        pltpu.VMEM((ISLOTS, CHUNK), jnp.int32),
        pltpu.VMEM((CHUNK, W), jnp.float32),
        pltpu.VMEM_SHARED((NP, W), jnp.float32),
        pltpu.SemaphoreType.DMA,
        pltpu.SemaphoreType.DMA,
    ],
    compiler_params=_UNTILED,
)
def _deg_kernel(dst_hbm, z_hbm, d0_hbm, d1_hbm, didx, ones, dacc, isem, ssem):
    cid = lax.axis_index("c")
    sid = lax.axis_index("s")
    wid = cid * NS + sid
    row0 = sid * ROWS_PER_TILE

    def fill(r, carry):
        for k in range(W // 16):
            ones[r, pl.ds(k * 16, 16)] = jnp.ones((16,), jnp.float32)
        return carry

    lax.fori_loop(0, CHUNK, fill, 0)
    pltpu.sync_copy(z_hbm, dacc.at[pl.ds(row0, ROWS_PER_TILE)])
    plsc.subcore_barrier()
    base = wid * EDGES_PER_TILE

    def body(c, carry):
        @pl.when(c >= 2)
        def _():
            cs = c - 2
            mi = lax.rem(cs, ISLOTS)
            pltpu.make_async_copy(
                dst_hbm.at[pl.ds(0, CHUNK)], didx.at[mi], isem
            ).wait()
            pltpu.async_copy(ones, dacc.at[didx.at[mi]], ssem, add=True)

            @pl.when(cs >= 6)
            def _():
                pltpu.make_async_copy(ones, dacc.at[didx.at[mi]], ssem).wait()

        @pl.when(c < NCHUNKS)
        def _():
            off = base + c * CHUNK
            pltpu.async_copy(
                dst_hbm.at[pl.ds(off, CHUNK)], didx.at[lax.rem(c, ISLOTS)], isem
            )
        return carry

    lax.fori_loop(0, NCHUNKS + 2, body, 0)
    for _ in range(6):
        pltpu.make_async_copy(ones, dacc.at[didx.at[0]], ssem).wait()
    plsc.subcore_barrier()

    @pl.when(cid == 0)
    def _():
        pltpu.sync_copy(
            dacc.at[pl.ds(row0, ROWS_PER_TILE)], d0_hbm.at[pl.ds(row0, ROWS_PER_TILE)]
        )

    @pl.when(cid == 1)
    def _():
        pltpu.sync_copy(
            dacc.at[pl.ds(row0, ROWS_PER_TILE)], d1_hbm.at[pl.ds(row0, ROWS_PER_TILE)]
        )


@functools.partial(
    pl.kernel,
    out_type=(
        jax.ShapeDtypeStruct((NP, W), jnp.float32),
        jax.ShapeDtypeStruct((NP, W), jnp.float32),
        jax.ShapeDtypeStruct((NP, W), jnp.float32),
    ),
    mesh=_MESH,
    scratch_types=[
        pltpu.VMEM((RR, 128), jnp.float32),
        pltpu.VMEM((RR, W), jnp.float32),
        pltpu.VMEM((RR, W), jnp.float32),
        pltpu.VMEM((RR, W), jnp.float32),
        pltpu.VMEM((RR, W), jnp.float32),
        pltpu.VMEM((RR, W), jnp.float32),
    ],
    compiler_params=_UNTILED,
)
def _prep_kernel(h128_hbm, d0_hbm, d1_hbm, h64_hbm, sc_hbm, a0_hbm,
                 hb, d0b, d1b, hob, scb, aob):
    cid = lax.axis_index("c")
    sid = lax.axis_index("s")
    row0 = sid * ROWS_PER_TILE

    @pl.when(cid == 0)
    def _():
        for r in range(NROUNDS):
            b = row0 + r * RR
            pltpu.sync_copy(h128_hbm.at[pl.ds(b, RR)], hb)
            pltpu.sync_copy(d0_hbm.at[pl.ds(b, RR)], d0b)
            pltpu.sync_copy(d1_hbm.at[pl.ds(b, RR)], d1b)

            def conv(j, carry):
                for k in range(W // 16):
                    s = pl.ds(k * 16, 16)
                    v = hb[j, s]
                    hob[j, s] = v
                    aob[j, s] = ALPHA * v
                    dv = d0b[j, s] + d1b[j, s]
                    scb[j, s] = (1.0 - ALPHA) / jnp.maximum(dv, 1.0)
                return carry

            lax.fori_loop(0, RR, conv, 0)
            pltpu.sync_copy(hob, h64_hbm.at[pl.ds(b, RR)])
            pltpu.sync_copy(scb, sc_hbm.at[pl.ds(b, RR)])
            pltpu.sync_copy(aob, a0_hbm.at[pl.ds(b, RR)])


@functools.partial(
    pl.kernel,
    out_type=(
        jax.ShapeDtypeStruct((NP, W), jnp.float32),
        jax.ShapeDtypeStruct((NP, W), jnp.float32),
    ),
    mesh=_MESH,
    scratch_types=[
        pltpu.VMEM((ISLOTS, CHUNK), jnp.int32),
        pltpu.VMEM((ISLOTS, CHUNK), jnp.int32),
        pltpu.VMEM((MSLOTS, CHUNK, W), jnp.float32),
        pltpu.VMEM_SHARED((NP, W), jnp.float32),
        pltpu.VMEM_SHARED((NP, W), jnp.float32),
        pltpu.SemaphoreType.DMA,
        pltpu.SemaphoreType.DMA,
        pltpu.SemaphoreType.DMA,
    ],
    compiler_params=_UNTILED,
)
def _edge_kernel(h_hbm, src_hbm, dst_hbm, z_hbm, p0_hbm, p1_hbm,
                 sidx, didx, msg, tab, acc, isem, gsem, ssem):
    cid = lax.axis_index("c")
    sid = lax.axis_index("s")
    wid = cid * NS + sid
    row0 = sid * ROWS_PER_TILE
    pltpu.sync_copy(
        h_hbm.at[pl.ds(row0, ROWS_PER_TILE)], tab.at[pl.ds(row0, ROWS_PER_TILE)]
    )
    pltpu.sync_copy(z_hbm, acc.at[pl.ds(row0, ROWS_PER_TILE)])
    plsc.subcore_barrier()
    base = wid * EDGES_PER_TILE

    def body(c, carry):
        @pl.when(c >= 3)
        def _():
            cs = c - 3
            ms = lax.rem(cs, MSLOTS)
            mi = lax.rem(cs, ISLOTS)
            pltpu.make_async_copy(tab.at[sidx.at[mi]], msg.at[ms], gsem).wait()
            pltpu.async_copy(msg.at[ms], acc.at[didx.at[mi]], ssem, add=True)

        @pl.when(jnp.logical_and(c >= 2, c < NCHUNKS + 2))
        def _():
            cg = c - 2
            mg = lax.rem(cg, MSLOTS)
            ig = lax.rem(cg, ISLOTS)

            @pl.when(c >= 2 + MSLOTS)
            def _():
                pltpu.make_async_copy(
                    msg.at[mg], acc.at[didx.at[ig]], ssem
                ).wait()

            pltpu.make_async_copy(
                src_hbm.at[pl.ds(0, CHUNK)], sidx.at[ig], isem
            ).wait()
            pltpu.make_async_copy(
                dst_hbm.at[pl.ds(0, CHUNK)], didx.at[ig], isem
            ).wait()
            pltpu.async_copy(tab.at[sidx.at[ig]], msg.at[mg], gsem)

        @pl.when(c < NCHUNKS)
        def _():
            off = base + c * CHUNK
            ii = lax.rem(c, ISLOTS)
            pltpu.async_copy(src_hbm.at[pl.ds(off, CHUNK)], sidx.at[ii], isem)
            pltpu.async_copy(dst_hbm.at[pl.ds(off, CHUNK)], didx.at[ii], isem)
        return carry

    lax.fori_loop(0, NCHUNKS + 3, body, 0)
    for _ in range(MSLOTS):
        pltpu.make_async_copy(msg.at[0], acc.at[didx.at[0]], ssem).wait()
    plsc.subcore_barrier()

    @pl.when(cid == 0)
    def _():
        pltpu.sync_copy(
            acc.at[pl.ds(row0, ROWS_PER_TILE)], p0_hbm.at[pl.ds(row0, ROWS_PER_TILE)]
        )

    @pl.when(cid == 1)
    def _():
        pltpu.sync_copy(
            acc.at[pl.ds(row0, ROWS_PER_TILE)], p1_hbm.at[pl.ds(row0, ROWS_PER_TILE)]
        )


@functools.partial(
    pl.kernel,
    out_type=jax.ShapeDtypeStruct((NP, W), jnp.float32),
    mesh=_MESH,
    scratch_types=[
        pltpu.VMEM((RR, W), jnp.float32),
        pltpu.VMEM((RR, W), jnp.float32),
        pltpu.VMEM((RR, W), jnp.float32),
        pltpu.VMEM((RR, W), jnp.float32),
        pltpu.VMEM((RR, W), jnp.float32),
    ],
    compiler_params=_UNTILED,
)
def _mix_kernel(p0_hbm, p1_hbm, sc_hbm, a0_hbm, h_hbm, p0b, p1b, scb, aob, ob):
    cid = lax.axis_index("c")
    sid = lax.axis_index("s")
    row0 = sid * ROWS_PER_TILE

    @pl.when(cid == 0)
    def _():
        for r in range(NROUNDS):
            b = row0 + r * RR
            pltpu.sync_copy(p0_hbm.at[pl.ds(b, RR)], p0b)
            pltpu.sync_copy(p1_hbm.at[pl.ds(b, RR)], p1b)
            pltpu.sync_copy(sc_hbm.at[pl.ds(b, RR)], scb)
            pltpu.sync_copy(a0_hbm.at[pl.ds(b, RR)], aob)

            def conv(j, carry):
                for k in range(W // 16):
                    s = pl.ds(k * 16, 16)
                    ob[j, s] = (p0b[j, s] + p1b[j, s]) * scb[j, s] + aob[j, s]
                return carry

            lax.fori_loop(0, RR, conv, 0)
            pltpu.sync_copy(ob, h_hbm.at[pl.ds(b, RR)])


@functools.partial(
    pl.kernel,
    out_type=jax.ShapeDtypeStruct((HT_ROWS, 128), jnp.float32),
    mesh=_MESH,
    scratch_types=[
        pltpu.VMEM((RR, W), jnp.float32),
        pltpu.VMEM((RR // 2, 128), jnp.float32),
    ],
    compiler_params=_UNTILED,
)
def _to128_kernel(h_hbm, o_hbm, b64, b128):
    cid = lax.axis_index("c")
    sid = lax.axis_index("s")
    row0 = sid * ROWS_PER_TILE
    trow0 = sid * TROWS_PER_TILE

    @pl.when(cid == 0)
    def _():
        for r in range(NROUNDS):
            pltpu.sync_copy(h_hbm.at[pl.ds(row0 + r * RR, RR)], b64)

            def conv(t, carry):
                for k in range(8):
                    b128[t, pl.ds(k * 16, 16)] = b64[
                        2 * t + (1 if k >= 4 else 0), pl.ds((k % 4) * 16, 16)
                    ]
                return carry

            lax.fori_loop(0, RR // 2, conv, 0)
            pltpu.sync_copy(b128, o_hbm.at[pl.ds(trow0 + r * (RR // 2), RR // 2)])


def kernel(x, edge_index, W1, b1, W2, b2):
    ei = edge_index.astype(jnp.int32)
    src = ei[0]
    dst = ei[1]
    pad_e = EDGES_PAD - N_EDGES
    srcp = jnp.concatenate([src, jnp.zeros((pad_e,), jnp.int32)])
    dstp = jnp.concatenate([dst, jnp.full((pad_e,), NP - 1, jnp.int32)])

    xp = jnp.concatenate([x, jnp.zeros((NP - N_NODES, FEATS), jnp.float32)])
    h0p128 = _front(xp, W1, b1, W2, b2)

    z64 = jnp.zeros((ROWS_PER_TILE, W), jnp.float32)

    d0, d1 = _deg_kernel(dstp, z64)
    h, scale, ah0 = _prep_kernel(h0p128, d0, d1)

    for _ in range(DEPTH):
        p0, p1 = _edge_kernel(h, srcp, dstp, z64)
        h = _mix_kernel(p0, p1, scale, ah0)
    ht = _to128_kernel(h)
    return ht.reshape(NP, CLASSES)[:N_NODES]


# speedup vs baseline: 1.1151x; 1.1151x over previous
import functools

import jax
import jax.numpy as jnp
from jax import lax
from jax.experimental import pallas as pl
from jax.experimental.pallas import tpu as pltpu
from jax.experimental.pallas import tpu_sc as plsc

N_NODES = 10000
N_EDGES = 320000
FEATS = 128
HIDDEN = 64
CLASSES = 64
ALPHA = 0.1
DEPTH = 10

NC = 2
NS = 16
NW = NC * NS
CHUNK = 128
NCHUNKS = 80
EDGES_PER_TILE = NCHUNKS * CHUNK
EDGES_PAD = NW * EDGES_PER_TILE
NP = 10240
W = CLASSES
ROWS_PER_TILE = NP // NS
HT_ROWS = NP * W // 128
TROWS_PER_TILE = HT_ROWS // NS
RR = 128
NROUNDS = ROWS_PER_TILE // RR

MSLOTS = 4
ISLOTS = 8

_MESH = plsc.VectorSubcoreMesh(core_axis_name="c", subcore_axis_name="s")
_UNTILED = pltpu.CompilerParams(use_tc_tiling_on_sc=False)


def _front_body(x_ref, w1_ref, b1_ref, w2_ref, b2_ref, o_ref):
    h = jnp.dot(x_ref[...], w1_ref[...], preferred_element_type=jnp.float32)
    h = jnp.maximum(h + b1_ref[...], 0.0)
    h = jnp.dot(h, w2_ref[...], preferred_element_type=jnp.float32) + b2_ref[...]
    o_ref[...] = jnp.concatenate(
        [h, jnp.zeros((h.shape[0], 128 - CLASSES), jnp.float32)], axis=1
    )


def _front(xp, W1, b1, W2, b2):
    R = 1024
    return pl.pallas_call(
        _front_body,
        grid=(NP // R,),
        in_specs=[
            pl.BlockSpec((R, FEATS), lambda i: (i, 0)),
            pl.BlockSpec((FEATS, HIDDEN), lambda i: (0, 0)),
            pl.BlockSpec((1, HIDDEN), lambda i: (0, 0)),
            pl.BlockSpec((HIDDEN, CLASSES), lambda i: (0, 0)),
            pl.BlockSpec((1, CLASSES), lambda i: (0, 0)),
        ],
        out_specs=pl.BlockSpec((R, 128), lambda i: (i, 0)),
        out_shape=jax.ShapeDtypeStruct((NP, 128), jnp.float32),
    )(xp, W1, b1.reshape(1, HIDDEN), W2, b2.reshape(1, CLASSES))


@functools.partial(
    pl.kernel,
    out_type=(
        jax.ShapeDtypeStruct((NP, W), jnp.float32),
        jax.ShapeDtypeStruct((NP, W), jnp.float32),
    ),
    mesh=_MESH,
    scratch_types=[
        pltpu.VMEM((ISLOTS, CHUNK), jnp.int32),
        pltpu.VMEM((CHUNK, W), jnp.float32),
        pltpu.VMEM_SHARED((NP, W), jnp.float32),
        pltpu.SemaphoreType.DMA,
        pltpu.SemaphoreType.DMA,
    ],
    compiler_params=_UNTILED,
)
def _deg_kernel(dst_hbm, z_hbm, d0_hbm, d1_hbm, didx, ones, dacc, isem, ssem):
    cid = lax.axis_index("c")
    sid = lax.axis_index("s")
    wid = cid * NS + sid
    row0 = sid * ROWS_PER_TILE

    def fill(r, carry):
        for k in range(W // 16):
            ones[r, pl.ds(k * 16, 16)] = jnp.ones((16,), jnp.float32)
        return carry

    lax.fori_loop(0, CHUNK, fill, 0)
    pltpu.sync_copy(z_hbm, dacc.at[pl.ds(row0, ROWS_PER_TILE)])
    plsc.subcore_barrier()
    base = wid * EDGES_PER_TILE

    def body(c, carry):
        @pl.when(c >= 2)
        def _():
            cs = c - 2
            mi = lax.rem(cs, ISLOTS)
            pltpu.make_async_copy(
                dst_hbm.at[pl.ds(0, CHUNK)], didx.at[mi], isem
            ).wait()
            pltpu.async_copy(ones, dacc.at[didx.at[mi]], ssem, add=True)

            @pl.when(cs >= 6)
            def _():
                pltpu.make_async_copy(ones, dacc.at[didx.at[mi]], ssem).wait()

        @pl.when(c < NCHUNKS)
        def _():
            off = base + c * CHUNK
            pltpu.async_copy(
                dst_hbm.at[pl.ds(off, CHUNK)], didx.at[lax.rem(c, ISLOTS)], isem
            )
        return carry

    lax.fori_loop(0, NCHUNKS + 2, body, 0)
    for _ in range(6):
        pltpu.make_async_copy(ones, dacc.at[didx.at[0]], ssem).wait()
    plsc.subcore_barrier()

    @pl.when(cid == 0)
    def _():
        pltpu.sync_copy(
            dacc.at[pl.ds(row0, ROWS_PER_TILE)], d0_hbm.at[pl.ds(row0, ROWS_PER_TILE)]
        )

    @pl.when(cid == 1)
    def _():
        pltpu.sync_copy(
            dacc.at[pl.ds(row0, ROWS_PER_TILE)], d1_hbm.at[pl.ds(row0, ROWS_PER_TILE)]
        )


PREP_R = 160


@functools.partial(
    pl.kernel,
    out_type=(
        jax.ShapeDtypeStruct((NP, W), jnp.float32),
        jax.ShapeDtypeStruct((NP, 128), jnp.float32),
    ),
    mesh=_MESH,
    scratch_types=[
        pltpu.VMEM((PREP_R, 128), jnp.float32),
        pltpu.VMEM((PREP_R, W), jnp.float32),
        pltpu.VMEM((PREP_R, W), jnp.float32),
        pltpu.VMEM((PREP_R, W), jnp.float32),
        pltpu.VMEM((PREP_R, 128), jnp.float32),
    ],
    compiler_params=_UNTILED,
)
def _prep_kernel(h128_hbm, d0_hbm, d1_hbm, h64_hbm, sa_hbm,
                 hb, d0b, d1b, hob, sab):
    cid = lax.axis_index("c")
    sid = lax.axis_index("s")
    wid = cid * NS + sid
    row0 = wid * (NP // NW)

    for r in range(2):
        b = row0 + r * PREP_R
        pltpu.sync_copy(h128_hbm.at[pl.ds(b, PREP_R)], hb)
        pltpu.sync_copy(d0_hbm.at[pl.ds(b, PREP_R)], d0b)
        pltpu.sync_copy(d1_hbm.at[pl.ds(b, PREP_R)], d1b)

        def conv(j, carry):
            for k in range(W // 16):
                sl = pl.ds(k * 16, 16)
                v = hb[j, sl]
                hob[j, sl] = v
                sab[j, pl.ds(64 + k * 16, 16)] = ALPHA * v
                dv = d0b[j, sl] + d1b[j, sl]
                sab[j, sl] = (1.0 - ALPHA) / jnp.maximum(dv, 1.0)
            return carry

        lax.fori_loop(0, PREP_R, conv, 0)
        pltpu.sync_copy(hob, h64_hbm.at[pl.ds(b, PREP_R)])
        pltpu.sync_copy(sab, sa_hbm.at[pl.ds(b, PREP_R)])


@functools.partial(
    pl.kernel,
    out_type=(
        jax.ShapeDtypeStruct((NP, W), jnp.float32),
        jax.ShapeDtypeStruct((NP, W), jnp.float32),
    ),
    mesh=_MESH,
    scratch_types=[
        pltpu.VMEM((ISLOTS, CHUNK), jnp.int32),
        pltpu.VMEM((ISLOTS, CHUNK), jnp.int32),
        pltpu.VMEM((MSLOTS, CHUNK, W), jnp.float32),
        pltpu.VMEM_SHARED((NP, W), jnp.float32),
        pltpu.VMEM_SHARED((NP, W), jnp.float32),
        pltpu.SemaphoreType.DMA,
        pltpu.SemaphoreType.DMA,
        pltpu.SemaphoreType.DMA,
    ],
    compiler_params=_UNTILED,
)
def _edge_kernel(h_hbm, src_hbm, dst_hbm, z_hbm, p0_hbm, p1_hbm,
                 sidx, didx, msg, tab, acc, isem, gsem, ssem):
    cid = lax.axis_index("c")
    sid = lax.axis_index("s")
    wid = cid * NS + sid
    row0 = sid * ROWS_PER_TILE
    pltpu.sync_copy(
        h_hbm.at[pl.ds(row0, ROWS_PER_TILE)], tab.at[pl.ds(row0, ROWS_PER_TILE)]
    )
    pltpu.sync_copy(z_hbm, acc.at[pl.ds(row0, ROWS_PER_TILE)])
    plsc.subcore_barrier()
    base = wid * EDGES_PER_TILE

    def body(c, carry):
        @pl.when(c >= 3)
        def _():
            cs = c - 3
            ms = lax.rem(cs, MSLOTS)
            mi = lax.rem(cs, ISLOTS)
            pltpu.make_async_copy(tab.at[sidx.at[mi]], msg.at[ms], gsem).wait()
            pltpu.async_copy(msg.at[ms], acc.at[didx.at[mi]], ssem, add=True)

        @pl.when(jnp.logical_and(c >= 2, c < NCHUNKS + 2))
        def _():
            cg = c - 2
            mg = lax.rem(cg, MSLOTS)
            ig = lax.rem(cg, ISLOTS)

            @pl.when(c >= 2 + MSLOTS)
            def _():
                pltpu.make_async_copy(
                    msg.at[mg], acc.at[didx.at[ig]], ssem
                ).wait()

            pltpu.make_async_copy(
                src_hbm.at[pl.ds(0, CHUNK)], sidx.at[ig], isem
            ).wait()
            pltpu.make_async_copy(
                dst_hbm.at[pl.ds(0, CHUNK)], didx.at[ig], isem
            ).wait()
            pltpu.async_copy(tab.at[sidx.at[ig]], msg.at[mg], gsem)

        @pl.when(c < NCHUNKS)
        def _():
            off = base + c * CHUNK
            ii = lax.rem(c, ISLOTS)
            pltpu.async_copy(src_hbm.at[pl.ds(off, CHUNK)], sidx.at[ii], isem)
            pltpu.async_copy(dst_hbm.at[pl.ds(off, CHUNK)], didx.at[ii], isem)
        return carry

    lax.fori_loop(0, NCHUNKS + 3, body, 0)
    for _ in range(MSLOTS):
        pltpu.make_async_copy(msg.at[0], acc.at[didx.at[0]], ssem).wait()
    plsc.subcore_barrier()

    @pl.when(cid == 0)
    def _():
        pltpu.sync_copy(
            acc.at[pl.ds(row0, ROWS_PER_TILE)], p0_hbm.at[pl.ds(row0, ROWS_PER_TILE)]
        )

    @pl.when(cid == 1)
    def _():
        pltpu.sync_copy(
            acc.at[pl.ds(row0, ROWS_PER_TILE)], p1_hbm.at[pl.ds(row0, ROWS_PER_TILE)]
        )


MIX_R = NP // NW


@functools.partial(
    pl.kernel,
    out_type=jax.ShapeDtypeStruct((NP, W), jnp.float32),
    mesh=_MESH,
    scratch_types=[
        pltpu.VMEM((MIX_R, W), jnp.float32),
        pltpu.VMEM((MIX_R, W), jnp.float32),
        pltpu.VMEM((MIX_R, 128), jnp.float32),
        pltpu.VMEM((MIX_R, W), jnp.float32),
        pltpu.SemaphoreType.DMA,
    ],
    compiler_params=_UNTILED,
)
def _mix_kernel(p0_hbm, p1_hbm, sa_hbm, h_hbm, p0b, p1b, sab, ob, sem):
    cid = lax.axis_index("c")
    sid = lax.axis_index("s")
    wid = cid * NS + sid
    b = wid * MIX_R
    c0 = pltpu.async_copy(p0_hbm.at[pl.ds(b, MIX_R)], p0b, sem)
    c1 = pltpu.async_copy(p1_hbm.at[pl.ds(b, MIX_R)], p1b, sem)
    c2 = pltpu.async_copy(sa_hbm.at[pl.ds(b, MIX_R)], sab, sem)
    c0.wait()
    c1.wait()
    c2.wait()

    def conv(j, carry):
        for k in range(W // 16):
            sl = pl.ds(k * 16, 16)
            ob[j, sl] = (p0b[j, sl] + p1b[j, sl]) * sab[j, sl] + sab[
                j, pl.ds(64 + k * 16, 16)
            ]
        return carry

    lax.fori_loop(0, MIX_R, conv, 0)
    pltpu.sync_copy(ob, h_hbm.at[pl.ds(b, MIX_R)])


@functools.partial(
    pl.kernel,
    out_type=jax.ShapeDtypeStruct((HT_ROWS, 128), jnp.float32),
    mesh=_MESH,
    scratch_types=[
        pltpu.VMEM((MIX_R, W), jnp.float32),
        pltpu.VMEM((MIX_R // 2, 128), jnp.float32),
    ],
    compiler_params=_UNTILED,
)
def _to128_kernel(h_hbm, o_hbm, b64, b128):
    cid = lax.axis_index("c")
    sid = lax.axis_index("s")
    wid = cid * NS + sid
    pltpu.sync_copy(h_hbm.at[pl.ds(wid * MIX_R, MIX_R)], b64)

    def conv(t, carry):
        for k in range(8):
            b128[t, pl.ds(k * 16, 16)] = b64[
                2 * t + (1 if k >= 4 else 0), pl.ds((k % 4) * 16, 16)
            ]
        return carry

    lax.fori_loop(0, MIX_R // 2, conv, 0)
    pltpu.sync_copy(b128, o_hbm.at[pl.ds(wid * (MIX_R // 2), MIX_R // 2)])


def kernel(x, edge_index, W1, b1, W2, b2):
    ei = edge_index.astype(jnp.int32)
    src = ei[0]
    dst = ei[1]
    pad_e = EDGES_PAD - N_EDGES
    srcp = jnp.concatenate([src, jnp.zeros((pad_e,), jnp.int32)])
    dstp = jnp.concatenate([dst, jnp.full((pad_e,), NP - 1, jnp.int32)])

    xp = jnp.concatenate([x, jnp.zeros((NP - N_NODES, FEATS), jnp.float32)])
    h0p128 = _front(xp, W1, b1, W2, b2)

    z64 = jnp.zeros((ROWS_PER_TILE, W), jnp.float32)

    d0, d1 = _deg_kernel(dstp, z64)
    h, sa = _prep_kernel(h0p128, d0, d1)

    for _ in range(DEPTH):
        p0, p1 = _edge_kernel(h, srcp, dstp, z64)
        h = _mix_kernel(p0, p1, sa)
    ht = _to128_kernel(h)
    return ht.reshape(NP, CLASSES)[:N_NODES]
